# Initial kernel scaffold; baseline (speedup 1.0000x reference)
import jax
import jax.numpy as jnp
from jax.experimental import pallas as pl


def kernel(q_batch, q_indices, X, pre_indices, pre_weights, W, b):
    raise NotImplementedError("write your pallas kernel here")


# speedup vs baseline: 4.3771x; 4.3771x over previous
import functools

import jax
import jax.numpy as jnp
from jax import lax
from jax.experimental import pallas as pl
from jax.experimental.pallas import tpu as pltpu
from jax.experimental.pallas import tpu_sc as plsc

_ALPHA = 1.0
_BETA = 1.0
_LAMB = 1e-4
_K = 16
_TAU = 0.1
_EPS = 0.05
_DELTA = 0.01
_SINK = 50

_B = 128
_D = 128
_U = 32
_TN = 2000
_NSTEP = 50
_NROW = 100000


def _dot(a, b, dims):
    return lax.dot_general(a, b, (dims, ((), ())),
                           preferred_element_type=jnp.float32)


def _eye(n, dtype=jnp.float32):
    r = lax.broadcasted_iota(jnp.int32, (n, n), 0)
    c = lax.broadcasted_iota(jnp.int32, (n, n), 1)
    return (r == c).astype(dtype)


def _transpose(a):
    n = a.shape[0]
    return _dot(a, _eye(n), ((0,), (0,)))


def _t_intsplit(a):
    c2 = jnp.floor(a * (1.0 / 65536.0))
    r = a - c2 * 65536.0
    c1 = jnp.floor(r * (1.0 / 256.0))
    c0 = r - c1 * 256.0
    return (_transpose(c0) + 256.0 * _transpose(c1)
            + 65536.0 * _transpose(c2))


def _t_f32(a):
    hi = a.astype(jnp.bfloat16).astype(jnp.float32)
    r1 = a - hi
    mid = r1.astype(jnp.bfloat16).astype(jnp.float32)
    lo = r1 - mid
    return _transpose(hi) + _transpose(mid) + _transpose(lo)


def _stage_a_body(praw_ref, wraw_ref, qi_ref, q_ref, w_ref, b_ref, x_ref,
                  post_ref, union_ref, invp_ref, invq_ref,
                  s_ref, sx_ref, t0_ref, prew_ref,
                  t0_scr, t0t_scr, s_scr, sx_scr, topv_scr, topi_scr, d_scr):
    step = pl.program_id(0)

    @pl.when(step == 0)
    def _init():
        t0 = _dot(q_ref[...], w_ref[...], ((1,), (1,))) + b_ref[...]
        t0_scr[...] = t0
        t0t_scr[...] = _transpose(t0)
        s_scr[...] = jnp.zeros_like(s_scr)
        sx_scr[...] = jnp.zeros_like(sx_scr)
        topv_scr[...] = jnp.full_like(topv_scr, jnp.inf)
        topi_scr[...] = jnp.zeros_like(topi_scr)

    x = x_ref[...]
    s_scr[...] += _dot(x, x, ((0,), (0,)))
    sx_scr[...] += jnp.sum(x, axis=0, keepdims=True)

    xn = jnp.sum(x * x, axis=1, keepdims=True)
    d_scr[...] = xn - 2.0 * _dot(x, t0t_scr[...], ((1,), (0,)))

    base = jnp.float32(step * _TN)
    slot = lax.broadcasted_iota(jnp.int32, (_K, _B), 0)
    big_f = jnp.float32(3e38)

    def cond(carry):
        return carry

    def body(carry):
        lane = (lax.broadcasted_iota(jnp.int32, (_TN, _B), 0)
                .astype(jnp.float32) + base)
        d = d_scr[...]
        topv = topv_scr[...]
        topi = topi_scr[...]
        minv = jnp.min(d, axis=0, keepdims=True)
        mini = jnp.min(jnp.where(d == minv, lane, big_f),
                       axis=0, keepdims=True)
        d_scr[...] = jnp.where(lane == mini, jnp.inf, d)
        beats = (topv < minv) | ((topv == minv) & (topi < mini))
        p = jnp.sum(beats.astype(jnp.int32), axis=0, keepdims=True)
        sv = jnp.concatenate([topv[:1], topv[:-1]], axis=0)
        si = jnp.concatenate([topi[:1], topi[:-1]], axis=0)
        bm = jnp.broadcast_to(minv, (_K, _B))
        bi = jnp.broadcast_to(mini, (_K, _B))
        topv_scr[...] = jnp.where(slot < p, topv,
                                  jnp.where(slot == p, bm, sv))
        topi_scr[...] = jnp.where(slot < p, topi,
                                  jnp.where(slot == p, bi, si))
        return jnp.any(p < _K)

    lax.while_loop(cond, body, jnp.bool_(True))

    @pl.when(step == _NSTEP - 1)
    def _finish():
        off = lax.rem(qi_ref[...], jnp.int32(8)) * _K
        coli = lax.broadcasted_iota(jnp.int32, (_B, _D), 1)
        pre_v, pre_w = [], []
        for j in range(_K):
            mask = coli == (off + j)
            pre_v.append(jnp.sum(jnp.where(mask, praw_ref[...], 0),
                                 axis=1, keepdims=True))
            pre_w.append(jnp.sum(jnp.where(mask, wraw_ref[...], 0.0),
                                 axis=1, keepdims=True))
        prew_ref[...] = jnp.concatenate(pre_w, axis=1)
        pre_bk = jnp.concatenate(pre_v, axis=1).astype(jnp.float32)
        pre_vals = _t_intsplit(pre_bk)

        cat = jnp.concatenate([pre_vals, topi_scr[...]], axis=0)
        ci = cat[:, None, :]
        cj = cat[None, :, :]
        ii = lax.broadcasted_iota(jnp.int32, (_U, _U, _B), 0)
        jj = lax.broadcasted_iota(jnp.int32, (_U, _U, _B), 1)
        eq = ci == cj
        dup_before = jnp.any(eq & (jj < ii), axis=1)
        first = jnp.logical_not(dup_before)
        less = cj < ci
        rank = jnp.sum((first[None, :, :] & less).astype(jnp.int32), axis=1)
        rank_f = rank.astype(jnp.float32)
        m = jnp.sum(first.astype(jnp.float32), axis=0, keepdims=True)
        minval = jnp.min(cat, axis=0, keepdims=True)
        rr = lax.broadcasted_iota(jnp.int32, (_U, _U, _B), 1)
        sel = first[:, None, :] & (rank[:, None, :] == rr)
        union0 = jnp.sum(jnp.where(sel, ci, 0.0), axis=0)
        r1 = (lax.broadcasted_iota(jnp.int32, (_U, _B), 0)
              .astype(jnp.float32))
        union_t = jnp.where(r1 < m, union0,
                            jnp.broadcast_to(minval, (_U, _B)))
        half = jnp.float32(0.5)
        union_ref[...] = (_t_intsplit(union_t) + half).astype(jnp.int32)
        post_ref[...] = (_t_intsplit(topi_scr[...]) + half).astype(jnp.int32)
        invp_ref[...] = (_transpose(rank_f[:_K]) + half).astype(jnp.int32)
        invq_ref[...] = (_transpose(rank_f[_K:]) + half).astype(jnp.int32)
        s_ref[...] = s_scr[...]
        sx_ref[...] = sx_scr[...]
        t0_ref[...] = t0_scr[...]


def _stage_a(praw, wraw, qi2, q_batch, X, W, b2):
    out_shapes = (
        jax.ShapeDtypeStruct((_B, _K), jnp.int32),
        jax.ShapeDtypeStruct((_B, _U), jnp.int32),
        jax.ShapeDtypeStruct((_B, _K), jnp.int32),
        jax.ShapeDtypeStruct((_B, _K), jnp.int32),
        jax.ShapeDtypeStruct((_D, _D), jnp.float32),
        jax.ShapeDtypeStruct((1, _D), jnp.float32),
        jax.ShapeDtypeStruct((_B, _D), jnp.float32),
        jax.ShapeDtypeStruct((_B, _K), jnp.float32),
    )
    whole = lambda shape: pl.BlockSpec(shape, lambda i: (0, 0))
    return pl.pallas_call(
        _stage_a_body,
        grid=(_NSTEP,),
        in_specs=[
            whole((_B, _D)),
            whole((_B, _D)),
            whole((_B, 1)),
            whole((_B, _D)),
            whole((_D, _D)),
            whole((1, _D)),
            pl.BlockSpec((_TN, _D), lambda i: (i, 0)),
        ],
        out_specs=tuple(whole(s.shape) for s in out_shapes),
        out_shape=out_shapes,
        scratch_shapes=[
            pltpu.VMEM((_B, _D), jnp.float32),
            pltpu.VMEM((_D, _B), jnp.float32),
            pltpu.VMEM((_D, _D), jnp.float32),
            pltpu.VMEM((1, _D), jnp.float32),
            pltpu.VMEM((_K, _B), jnp.float32),
            pltpu.VMEM((_K, _B), jnp.float32),
            pltpu.VMEM((_TN, _B), jnp.float32),
        ],
    )(praw, wraw, qi2, q_batch, W, b2, X)


def _sc_gather_pre(q_indices, pidx_view, pw_view):
    info = plsc.get_sparse_core_info()
    nc, ns = info.num_cores, info.num_subcores
    workers = 8
    rows = _B // workers

    mesh = plsc.VectorSubcoreMesh(core_axis_name="c", subcore_axis_name="s")

    @functools.partial(
        pl.kernel, mesh=mesh,
        out_type=(jax.ShapeDtypeStruct((_B, _D), jnp.int32),
                  jax.ShapeDtypeStruct((_B, _D), jnp.float32)),
        scratch_types=[
            pltpu.VMEM((rows,), jnp.int32),
            pltpu.VMEM((rows,), jnp.int32),
            pltpu.VMEM((rows, _D), jnp.int32),
            pltpu.VMEM((rows, _D), jnp.float32),
            pltpu.SemaphoreType.DMA,
        ],
    )
    def k(qi_hbm, pidx_hbm, pw_hbm, oi_hbm, ow_hbm,
          idx_v, idxb_v, ri_v, rw_v, sem):
        wid = lax.axis_index("s") * nc + lax.axis_index("c")

        @pl.when(wid < workers)
        def _():
            base = wid * rows
            pltpu.sync_copy(qi_hbm.at[pl.ds(base, rows)], idx_v)
            idxb_v[...] = lax.div(idx_v[...], jnp.int32(8))
            pltpu.async_copy(pidx_hbm.at[idxb_v], ri_v, sem).wait()
            pltpu.async_copy(pw_hbm.at[idxb_v], rw_v, sem).wait()
            pltpu.sync_copy(ri_v, oi_hbm.at[pl.ds(base, rows)])
            pltpu.sync_copy(rw_v, ow_hbm.at[pl.ds(base, rows)])

    return k(q_indices, pidx_view, pw_view)


def _sc_gather_rows(X, idx_all):
    info = plsc.get_sparse_core_info()
    nc, ns = info.num_cores, info.num_subcores
    nw = nc * ns
    total = idx_all.shape[0]
    per_w = total // nw
    chunk = per_w // 2

    mesh = plsc.VectorSubcoreMesh(core_axis_name="c", subcore_axis_name="s")

    @functools.partial(
        pl.kernel, mesh=mesh,
        out_type=jax.ShapeDtypeStruct((total, _D), jnp.float32),
        scratch_types=[
            pltpu.VMEM((chunk,), jnp.int32),
            pltpu.VMEM((chunk,), jnp.int32),
            pltpu.VMEM((chunk, _D), jnp.float32),
            pltpu.VMEM((chunk, _D), jnp.float32),
            pltpu.SemaphoreType.DMA,
            pltpu.SemaphoreType.DMA,
        ],
    )
    def k(x_hbm, idx_hbm, out_hbm, ia_v, ib_v, ra_v, rb_v, sa, sb):
        wid = lax.axis_index("s") * nc + lax.axis_index("c")
        base = wid * per_w
        pltpu.sync_copy(idx_hbm.at[pl.ds(base, chunk)], ia_v)
        pltpu.sync_copy(idx_hbm.at[pl.ds(base + chunk, chunk)], ib_v)
        ca = pltpu.async_copy(x_hbm.at[ia_v], ra_v, sa)
        cb = pltpu.async_copy(x_hbm.at[ib_v], rb_v, sb)
        ca.wait()
        pltpu.sync_copy(ra_v, out_hbm.at[pl.ds(base, chunk)])
        cb.wait()
        pltpu.sync_copy(rb_v, out_hbm.at[pl.ds(base + chunk, chunk)])

    return k(X, idx_all)


def _dot_hi(a, b):
    return lax.dot_general(a, b, ((((1,), (0,))), ((), ())),
                           preferred_element_type=jnp.float32,
                           precision=lax.Precision.HIGHEST)


def _ns_sqrt(a, iters):
    eye = _eye(_D)
    c = jnp.sqrt(jnp.sum(a * a))
    y0 = a / c

    y, z = y0, eye
    for _ in range(iters):
        t = 1.5 * eye - 0.5 * _dot_hi(z, y)
        y, z = _dot_hi(y, t), _dot_hi(t, z)
    return y * jnp.sqrt(c)


def _stage_c_body(t0_ref, s_ref, sx_ref, xnb_ref, sup_ref, prew_ref,
                  invp_ref, invq_ref, w_ref, b_ref,
                  tot_ref, dist_ref, knn_ref,
                  c_scr, km_scr, supt_scr):
    f32 = jnp.float32
    eye = _eye(_D)

    n = jnp.float32(_NROW)
    mu_x = sx_ref[...] / n
    sigx = (s_ref[...] - n * _dot(mu_x, mu_x, ((0,), (0,)))) / (n - 1.0)
    trx = jnp.sum(sigx * eye)
    sigx = sigx + (trx / _D * _DELTA) * eye
    sigx = (sigx + _t_f32(sigx)) * 0.5

    t0 = t0_ref[...]
    mu_t = jnp.sum(t0, axis=0, keepdims=True) / _B
    tc = t0 - mu_t
    sigt = _dot(tc, tc, ((0,), (0,))) / (_B - 1.0)
    trt = jnp.sum(sigt * eye)
    sigt = sigt + (trt / _D * _DELTA) * eye
    sigt = (sigt + _t_f32(sigt)) * 0.5

    term_mean = jnp.sum((mu_t - mu_x) ** 2)
    sqrt_t = _ns_sqrt(sigt, 30)
    m_mat = _dot(_dot(sqrt_t, sigx, ((1,), (0,))), sqrt_t, ((1,), (0,)))
    m_mat = (m_mat + _t_f32(m_mat)) * 0.5
    sqrt_m = _ns_sqrt(m_mat, 34)
    term_cov = (jnp.sum(sigx * eye) + jnp.sum(sigt * eye)
                - 2.0 * jnp.sum(sqrt_m * eye))
    loss_dist = term_mean + jnp.maximum(term_cov, 0.0)

    xnb = xnb_ref[...]
    diff = t0[:, None, :] - xnb
    l2 = jnp.sum(diff * diff, axis=-1)
    logits = -l2 / _TAU
    logits = logits - jnp.max(logits, axis=1, keepdims=True)
    e = jnp.exp(logits)
    post_w = e / jnp.sum(e, axis=1, keepdims=True)
    post_w = jnp.clip(post_w, 1e-8, None)
    post_w = post_w / jnp.sum(post_w, axis=1, keepdims=True)

    pre_w = jnp.clip(prew_ref[...], 1e-8, None)
    pre_w = pre_w / jnp.sum(pre_w, axis=1, keepdims=True)

    ip = invp_ref[...]
    kk1 = lax.broadcasted_iota(jnp.int32, (_B, _K, _K), 1)
    kk2 = lax.broadcasted_iota(jnp.int32, (_B, _K, _K), 2)
    eq_p = ip[:, :, None] == ip[:, None, :]
    dup_after = jnp.any(eq_p & (kk2 > kk1), axis=2)
    is_last = jnp.logical_not(dup_after)

    r_i = lax.broadcasted_iota(jnp.int32, (_B, _K, _U), 2)
    sel_p = (ip[:, :, None] == r_i) & is_last[:, :, None]
    p_br = jnp.sum(jnp.where(sel_p, pre_w[:, :, None], 0.0), axis=1)
    iq = invq_ref[...]
    sel_q = iq[:, :, None] == r_i
    q_br = jnp.sum(jnp.where(sel_q, post_w[:, :, None], 0.0), axis=1)

    p_t = _t_f32(p_br)
    q_t = _t_f32(q_br)

    for u in range(_U):
        supt_scr[u] = _t_f32(sup_ref[:, u, :])
    sup_t = supt_scr[...]
    for u in range(_U):
        dfu = sup_t - sup_t[u][None]
        c_scr[u] = jnp.sum(dfu * dfu, axis=1)

    cmat = c_scr[...]

    cbits = lax.bitcast_convert_type(cmat, jnp.int32)

    def os_bits(k_target):
        def bbody(_, lohi):
            lo, hi = lohi
            mid = lo + lax.div(hi - lo, 2)
            cnt = jnp.sum((cbits <= mid[None, None, :]).astype(jnp.int32),
                          axis=(0, 1))
            ge = cnt >= k_target
            return (jnp.where(ge, lo, mid + 1), jnp.where(ge, mid, hi))
        lo = jnp.zeros((_B,), jnp.int32)
        hi = jnp.full((_B,), 2139095040, jnp.int32)
        lo, hi = lax.fori_loop(0, 31, bbody, (lo, hi))
        return lo

    o1 = lax.bitcast_convert_type(os_bits(512), f32)
    o2 = lax.bitcast_convert_type(os_bits(513), f32)
    med = (o1 + o2) * 0.5
    cn = cmat / (med[None, None, :] + 1e-8)
    km_scr[...] = jnp.exp(-cn / _EPS)

    km = km_scr[...]

    def sbody(_, uv):
        uu, vv = uv
        kv = jnp.sum(km * vv[None, :, :], axis=1)
        uu = p_t / (kv + 1e-16)
        ku = jnp.sum(km * uu[:, None, :], axis=0)
        vv = q_t / (ku + 1e-16)
        return (uu, vv)

    ones_ub = jnp.ones((_U, _B), f32)
    uu, vv = lax.fori_loop(0, _SINK, sbody, (ones_ub, ones_ub))

    w2 = jnp.sum(uu[:, None, :] * km * cn * vv[None, :, :], axis=(0, 1))
    loss_knn = jnp.sum(w2) / _B

    loss_reg = (jnp.sum(w_ref[...] ** 2) + jnp.sum(b_ref[...] ** 2)) * 0.5
    total = _ALPHA * loss_dist + _BETA * loss_knn + _LAMB * loss_reg

    tot_ref[...] = jnp.broadcast_to(total, (1, 1))
    dist_ref[...] = jnp.broadcast_to(loss_dist, (1, 1))
    knn_ref[...] = jnp.broadcast_to(loss_knn, (1, 1))


def _stage_c(t0, s, sx, xnb, sup, pre_w_b, inv_pre, inv_post, W, b2):
    out_shapes = tuple(jax.ShapeDtypeStruct((1, 1), jnp.float32)
                       for _ in range(3))
    return pl.pallas_call(
        _stage_c_body,
        out_shape=out_shapes,
        scratch_shapes=[
            pltpu.VMEM((_U, _U, _B), jnp.float32),
            pltpu.VMEM((_U, _U, _B), jnp.float32),
            pltpu.VMEM((_U, _D, _B), jnp.float32),
        ],
    )(t0, s, sx, xnb, sup, pre_w_b, inv_pre, inv_post, W, b2)


def kernel(q_batch, q_indices, X, pre_indices, pre_weights, W, b):
    b2 = b.reshape(1, _D)
    praw, wraw = _sc_gather_pre(q_indices,
                                pre_indices.reshape(-1, _D),
                                pre_weights.reshape(-1, _D))
    post, union, inv_pre, inv_post, s_acc, sx, t0, pre_w_b = _stage_a(
        praw, wraw, q_indices.reshape(_B, 1), q_batch, X, W, b2)
    idx_all = jnp.concatenate([post.reshape(-1), union.reshape(-1)])
    rows = _sc_gather_rows(X, idx_all)
    xnb = rows[:_B * _K].reshape(_B, _K, _D)
    sup = rows[_B * _K:].reshape(_B, _U, _D)
    tot, dist, knn = _stage_c(t0, s_acc, sx, xnb, sup, pre_w_b,
                              inv_pre, inv_post, W, b2)
    return tot[0, 0], dist[0, 0], knn[0, 0]
